# baseline (speedup 1.0000x reference)
import dataclasses
import functools

import jax
import jax.numpy as jnp
from jax import lax
from jax.experimental import pallas as pl
from jax.experimental.pallas import tpu as pltpu
from jax.experimental.pallas import tpu_sc as plsc

N = 10000
E = 320000
D = 128

NC = 2
NS = 16
NW = NC * NS
EPW = 10240
EPAD = NW * EPW
K = 80
NCHUNK = EPW // K
NBUF = 2
NPAD = 10240
RPS = NPAD // NS

_mesh = plsc.VectorSubcoreMesh(core_axis_name="c", subcore_axis_name="s")

_sc_params = pltpu.CompilerParams()
if "needs_layout_passes" in pltpu.CompilerParams.__dataclass_fields__:
    _sc_params = dataclasses.replace(_sc_params, needs_layout_passes=False)


@functools.partial(
    pl.kernel,
    out_type=jax.ShapeDtypeStruct((NC, NPAD, D), jnp.float32),
    mesh=_mesh,
    compiler_params=_sc_params,
    scratch_types=[
        pltpu.VMEM_SHARED((NPAD, D), jnp.float32),
        pltpu.VMEM((EPW,), jnp.int32),
        pltpu.VMEM((NBUF, K), jnp.int32),
        pltpu.VMEM((EPW,), jnp.float32),
        pltpu.VMEM((NBUF, K, D), jnp.float32),
        pltpu.SemaphoreType.DMA((NBUF,)),
        pltpu.SemaphoreType.DMA((NBUF,)),
        pltpu.SemaphoreType.DMA((NBUF,)),
    ],
)
def _sc_aggregate(x_hbm, src_hbm, dst_hbm, w_hbm, out_hbm,
                  acc, src_v, dst_r, w_v, rows_v, gsem, ssem, dsem):
    cid = lax.axis_index("c")
    sid = lax.axis_index("s")
    wid = sid * NC + cid

    zbuf = rows_v.at[0]

    @pl.loop(0, K)
    def _zrow(i):
        for c in range(0, D, 16):
            zbuf[i, pl.ds(c, 16)] = jnp.zeros((16,), jnp.float32)

    @pl.loop(0, RPS, step=K)
    def _zcopy(r):
        pltpu.sync_copy(zbuf, acc.at[pl.ds(sid * RPS + r, K)])

    pltpu.sync_copy(src_hbm.at[wid], src_v)
    pltpu.sync_copy(w_hbm.at[wid], w_v)

    plsc.subcore_barrier()

    def gather_start(jj, t):
        pltpu.async_copy(dst_hbm.at[wid, jj], dst_r.at[t], dsem.at[t])
        pltpu.async_copy(x_hbm.at[src_v.at[pl.ds(jj * K, K)]],
                         rows_v.at[t], gsem.at[t])

    def gather_wait(jj, t):
        pltpu.make_async_copy(x_hbm.at[src_v.at[pl.ds(jj * K, K)]],
                              rows_v.at[t], gsem.at[t]).wait()

    def scatter_start(jj, t):
        pltpu.make_async_copy(dst_hbm.at[wid, jj], dst_r.at[t],
                              dsem.at[t]).wait()
        pltpu.async_copy(rows_v.at[t], acc.at[dst_r.at[t]], ssem.at[t],
                         add=True)

    def scatter_wait(jj, t):
        pltpu.make_async_copy(rows_v.at[t], acc.at[dst_r.at[t]],
                              ssem.at[t]).wait()

    for t in range(NBUF):
        gather_start(t, t)

    @pl.loop(0, NCHUNK, step=NBUF)
    def _grp(j):
        for t in range(NBUF):
            jj = j + t
            gather_wait(jj, t)

            @pl.loop(0, K, step=4)
            def _scale(i):
                for u in range(4):
                    wv = plsc.load_gather(
                        w_v, [jnp.full((16,), jj * K + i + u, jnp.int32)])
                    for c in range(0, D, 16):
                        rows_v[t, i + u, pl.ds(c, 16)] = (
                            rows_v[t, i + u, pl.ds(c, 16)] * wv)

            scatter_start(jj, t)

            pt = (t - 1) % NBUF
            pj = jj - 1

            @pl.when(jnp.logical_and(pj >= 0, pj + NBUF < NCHUNK))
            def _refill():
                scatter_wait(pj, pt)
                gather_start(pj + NBUF, pt)

    for t in range(NBUF):
        scatter_wait(NCHUNK - NBUF + t, t)

    plsc.subcore_barrier()

    pltpu.sync_copy(acc.at[pl.ds(sid * RPS, RPS)],
                    out_hbm.at[cid, pl.ds(sid * RPS, RPS)])


_BLK = 2000


def _gru_body(p_ref, x_ref, wih_ref, whh_ref, bih_ref, bhh_ref, o_ref):
    aggr = p_ref[0] + p_ref[1]
    x = x_ref[...]
    gi = jnp.dot(aggr, wih_ref[...], precision=lax.Precision.HIGHEST,
                 preferred_element_type=jnp.float32) + bih_ref[...]
    gh = jnp.dot(x, whh_ref[...], precision=lax.Precision.HIGHEST,
                 preferred_element_type=jnp.float32) + bhh_ref[...]
    r = jax.nn.sigmoid(gi[:, :D] + gh[:, :D])
    z = jax.nn.sigmoid(gi[:, D:2 * D] + gh[:, D:2 * D])
    n = jnp.tanh(gi[:, 2 * D:] + r * gh[:, 2 * D:])
    o_ref[...] = (1.0 - z) * n + z * x


def _tc_gru(p, x, wihT, whhT, bih, bhh):
    return pl.pallas_call(
        _gru_body,
        out_shape=jax.ShapeDtypeStruct((N, D), jnp.float32),
        grid=(N // _BLK,),
        in_specs=[
            pl.BlockSpec((NC, _BLK, D), lambda i: (0, i, 0)),
            pl.BlockSpec((_BLK, D), lambda i: (i, 0)),
            pl.BlockSpec((D, 3 * D), lambda i: (0, 0)),
            pl.BlockSpec((D, 3 * D), lambda i: (0, 0)),
            pl.BlockSpec((1, 3 * D), lambda i: (0, 0)),
            pl.BlockSpec((1, 3 * D), lambda i: (0, 0)),
        ],
        out_specs=pl.BlockSpec((_BLK, D), lambda i: (i, 0)),
    )(p, x, wihT, whhT, bih, bhh)


def kernel(x, edge_index, edge_weight,
           W_ih1, W_hh1, b_ih1, b_hh1, W_ih2, W_hh2, b_ih2, b_hh2):
    pad = EPAD - E
    src = jnp.concatenate(
        [edge_index[0], jnp.zeros((pad,), jnp.int32)]).reshape(NW, EPW)
    dst = jnp.concatenate(
        [edge_index[1], jnp.zeros((pad,), jnp.int32)]).reshape(NW, NCHUNK, K)
    w = jnp.concatenate(
        [edge_weight[:, 0], jnp.zeros((pad,), jnp.float32)]).reshape(NW, EPW)

    wih1T, whh1T = W_ih1.T, W_hh1.T
    wih2T, whh2T = W_ih2.T, W_hh2.T
    bih1, bhh1 = b_ih1.reshape(1, 3 * D), b_hh1.reshape(1, 3 * D)
    bih2, bhh2 = b_ih2.reshape(1, 3 * D), b_hh2.reshape(1, 3 * D)

    p1 = _sc_aggregate(x, src, dst, w)
    h = _tc_gru(p1, x, wih1T, whh1T, bih1, bhh1)
    p2 = _sc_aggregate(h, src, dst, w)
    return _tc_gru(p2, h, wih2T, whh2T, bih2, bhh2)


# speedup vs baseline: 3.4754x; 3.4754x over previous
import dataclasses
import functools

import jax
import jax.numpy as jnp
from jax import lax
from jax.experimental import pallas as pl
from jax.experimental.pallas import tpu as pltpu
from jax.experimental.pallas import tpu_sc as plsc

N = 10000
E = 320000
D = 128

NC = 2
NS = 16
NW = NC * NS
EPW = 10000
EPAD = NW * EPW
K = 40
NCHUNK = EPW // K
NBUF = 5
KA = 24
NPAD = 10240
RPS = NPAD // NS

_mesh = plsc.VectorSubcoreMesh(core_axis_name="c", subcore_axis_name="s")

_sc_params = pltpu.CompilerParams()
if "needs_layout_passes" in pltpu.CompilerParams.__dataclass_fields__:
    _sc_params = dataclasses.replace(_sc_params, needs_layout_passes=False)


@functools.partial(
    pl.kernel,
    out_type=jax.ShapeDtypeStruct((NC, NPAD, D), jnp.float32),
    mesh=_mesh,
    compiler_params=_sc_params,
    scratch_types=[
        pltpu.VMEM_SHARED((NPAD, D), jnp.float32),
        pltpu.VMEM((EPW,), jnp.int32),
        pltpu.VMEM((NBUF, K), jnp.int32),
        pltpu.VMEM((EPW,), jnp.float32),
        pltpu.VMEM((NBUF, K, D), jnp.float32),
        pltpu.SemaphoreType.DMA((NBUF,)),
        pltpu.SemaphoreType.DMA((NBUF,)),
        pltpu.SemaphoreType.DMA((NBUF,)),
        pltpu.SemaphoreType.DMA((NBUF,)),
    ],
)
def _sc_aggregate(x_hbm, src_hbm, dst_hbm, w_hbm, out_hbm,
                  acc, src_v, dst_r, w_v, rows_v, gsem, gsem2, ssem, dsem):
    cid = lax.axis_index("c")
    sid = lax.axis_index("s")
    wid = sid * NC + cid

    zbuf = rows_v.at[0]

    @pl.loop(0, K)
    def _zrow(i):
        for c in range(0, D, 16):
            zbuf[i, pl.ds(c, 16)] = jnp.zeros((16,), jnp.float32)

    @pl.loop(0, RPS, step=K)
    def _zcopy(r):
        pltpu.sync_copy(zbuf, acc.at[pl.ds(sid * RPS + r, K)])

    pltpu.sync_copy(src_hbm.at[wid], src_v)
    pltpu.sync_copy(w_hbm.at[wid], w_v)

    plsc.subcore_barrier()

    def gather_start(jj, t):
        pltpu.async_copy(dst_hbm.at[wid, jj], dst_r.at[t], dsem.at[t])
        pltpu.async_copy(x_hbm.at[src_v.at[pl.ds(jj * K, KA)]],
                         rows_v.at[t].at[pl.ds(0, KA)], gsem.at[t])
        pltpu.async_copy(x_hbm.at[src_v.at[pl.ds(jj * K + KA, K - KA)]],
                         rows_v.at[t].at[pl.ds(KA, K - KA)], gsem2.at[t])

    def gather_wait(jj, t):
        pltpu.make_async_copy(x_hbm.at[src_v.at[pl.ds(jj * K, KA)]],
                              rows_v.at[t].at[pl.ds(0, KA)],
                              gsem.at[t]).wait()
        pltpu.make_async_copy(x_hbm.at[src_v.at[pl.ds(jj * K + KA, K - KA)]],
                              rows_v.at[t].at[pl.ds(KA, K - KA)],
                              gsem2.at[t]).wait()

    def scatter_start(jj, t):
        pltpu.make_async_copy(dst_hbm.at[wid, jj], dst_r.at[t],
                              dsem.at[t]).wait()
        pltpu.async_copy(rows_v.at[t], acc.at[dst_r.at[t]], ssem.at[t],
                         add=True)

    def scatter_wait(jj, t):
        pltpu.make_async_copy(rows_v.at[t], acc.at[dst_r.at[t]],
                              ssem.at[t]).wait()

    for t in range(NBUF):
        gather_start(t, t)

    @pl.loop(0, NCHUNK, step=NBUF)
    def _grp(j):
        for t in range(NBUF):
            jj = j + t
            gather_wait(jj, t)

            @pl.loop(0, K, step=4)
            def _scale(i):
                for u in range(4):
                    wv = plsc.load_gather(
                        w_v, [jnp.full((16,), jj * K + i + u, jnp.int32)])
                    for c in range(0, D, 16):
                        rows_v[t, i + u, pl.ds(c, 16)] = (
                            rows_v[t, i + u, pl.ds(c, 16)] * wv)

            scatter_start(jj, t)

            pt = (t - 1) % NBUF
            pj = jj - 1

            @pl.when(jnp.logical_and(pj >= 0, pj + NBUF < NCHUNK))
            def _refill():
                scatter_wait(pj, pt)
                gather_start(pj + NBUF, pt)

    for t in range(NBUF):
        scatter_wait(NCHUNK - NBUF + t, t)

    plsc.subcore_barrier()

    pltpu.sync_copy(acc.at[pl.ds(sid * RPS, RPS)],
                    out_hbm.at[cid, pl.ds(sid * RPS, RPS)])


_BLK = 2000


def _gru_body(p_ref, x_ref, wih_ref, whh_ref, bih_ref, bhh_ref, o_ref):
    aggr = p_ref[0] + p_ref[1]
    x = x_ref[...]
    gi = jnp.dot(aggr, wih_ref[...], precision=lax.Precision.HIGHEST,
                 preferred_element_type=jnp.float32) + bih_ref[...]
    gh = jnp.dot(x, whh_ref[...], precision=lax.Precision.HIGHEST,
                 preferred_element_type=jnp.float32) + bhh_ref[...]
    r = jax.nn.sigmoid(gi[:, :D] + gh[:, :D])
    z = jax.nn.sigmoid(gi[:, D:2 * D] + gh[:, D:2 * D])
    n = jnp.tanh(gi[:, 2 * D:] + r * gh[:, 2 * D:])
    o_ref[...] = (1.0 - z) * n + z * x


def _tc_gru(p, x, wihT, whhT, bih, bhh):
    return pl.pallas_call(
        _gru_body,
        out_shape=jax.ShapeDtypeStruct((N, D), jnp.float32),
        grid=(N // _BLK,),
        in_specs=[
            pl.BlockSpec((NC, _BLK, D), lambda i: (0, i, 0)),
            pl.BlockSpec((_BLK, D), lambda i: (i, 0)),
            pl.BlockSpec((D, 3 * D), lambda i: (0, 0)),
            pl.BlockSpec((D, 3 * D), lambda i: (0, 0)),
            pl.BlockSpec((1, 3 * D), lambda i: (0, 0)),
            pl.BlockSpec((1, 3 * D), lambda i: (0, 0)),
        ],
        out_specs=pl.BlockSpec((_BLK, D), lambda i: (i, 0)),
    )(p, x, wihT, whhT, bih, bhh)


def kernel(x, edge_index, edge_weight,
           W_ih1, W_hh1, b_ih1, b_hh1, W_ih2, W_hh2, b_ih2, b_hh2):
    pad = EPAD - E
    src = jnp.concatenate(
        [edge_index[0], jnp.zeros((pad,), jnp.int32)]).reshape(NW, EPW)
    dst = jnp.concatenate(
        [edge_index[1], jnp.zeros((pad,), jnp.int32)]).reshape(NW, NCHUNK, K)
    w = jnp.concatenate(
        [edge_weight[:, 0], jnp.zeros((pad,), jnp.float32)]).reshape(NW, EPW)

    wih1T, whh1T = W_ih1.T, W_hh1.T
    wih2T, whh2T = W_ih2.T, W_hh2.T
    bih1, bhh1 = b_ih1.reshape(1, 3 * D), b_hh1.reshape(1, 3 * D)
    bih2, bhh2 = b_ih2.reshape(1, 3 * D), b_hh2.reshape(1, 3 * D)

    p1 = _sc_aggregate(x, src, dst, w)
    h = _tc_gru(p1, x, wih1T, whh1T, bih1, bhh1)
    p2 = _sc_aggregate(h, src, dst, w)
    return _tc_gru(p2, h, wih2T, whh2T, bih2, bhh2)


# speedup vs baseline: 3.5287x; 1.0153x over previous
import dataclasses
import functools

import jax
import jax.numpy as jnp
from jax import lax
from jax.experimental import pallas as pl
from jax.experimental.pallas import tpu as pltpu
from jax.experimental.pallas import tpu_sc as plsc

N = 10000
E = 320000
D = 128

NC = 2
NS = 16
NW = NC * NS
EPW = 10000
EPAD = NW * EPW
K = 40
NCHUNK = EPW // K
NBUF = 5
KA = 24
NPAD = 10240
RPS = NPAD // NS

_mesh = plsc.VectorSubcoreMesh(core_axis_name="c", subcore_axis_name="s")

_sc_params = pltpu.CompilerParams()
if "needs_layout_passes" in pltpu.CompilerParams.__dataclass_fields__:
    _sc_params = dataclasses.replace(_sc_params, needs_layout_passes=False)


@functools.partial(
    pl.kernel,
    out_type=jax.ShapeDtypeStruct((NC, NPAD, D), jnp.float32),
    mesh=_mesh,
    compiler_params=_sc_params,
    scratch_types=[
        pltpu.VMEM_SHARED((NPAD, D), jnp.float32),
        pltpu.VMEM((EPW,), jnp.int32),
        pltpu.VMEM((NBUF, K), jnp.int32),
        pltpu.VMEM((EPW,), jnp.float32),
        pltpu.VMEM((NBUF, K, D), jnp.float32),
        pltpu.SemaphoreType.DMA((NBUF,)),
        pltpu.SemaphoreType.DMA((NBUF,)),
        pltpu.SemaphoreType.DMA((NBUF,)),
        pltpu.SemaphoreType.DMA((NBUF,)),
    ],
)
def _sc_aggregate(x_hbm, src_hbm, dst_hbm, w_hbm, out_hbm,
                  acc, src_v, dst_r, w_v, rows_v, gsem, gsem2, ssem, dsem):
    cid = lax.axis_index("c")
    sid = lax.axis_index("s")
    wid = sid * NC + cid

    zbuf = rows_v.at[0]

    @pl.loop(0, K)
    def _zrow(i):
        for c in range(0, D, 16):
            zbuf[i, pl.ds(c, 16)] = jnp.zeros((16,), jnp.float32)

    pltpu.async_copy(src_hbm.at[wid], src_v, gsem.at[0])
    pltpu.async_copy(w_hbm.at[wid], w_v, gsem2.at[0])

    @pl.loop(0, RPS, step=K)
    def _zcopy(r):
        pltpu.async_copy(zbuf, acc.at[pl.ds(sid * RPS + r, K)], ssem.at[0])

    @pl.loop(0, RPS, step=K)
    def _zwait(r):
        pltpu.make_async_copy(zbuf, acc.at[pl.ds(sid * RPS + r, K)],
                              ssem.at[0]).wait()

    pltpu.make_async_copy(src_hbm.at[wid], src_v, gsem.at[0]).wait()
    pltpu.make_async_copy(w_hbm.at[wid], w_v, gsem2.at[0]).wait()

    plsc.subcore_barrier()

    def gather_start(jj, t):
        pltpu.async_copy(dst_hbm.at[wid, jj], dst_r.at[t], dsem.at[t])
        pltpu.async_copy(x_hbm.at[src_v.at[pl.ds(jj * K, KA)]],
                         rows_v.at[t].at[pl.ds(0, KA)], gsem.at[t])
        pltpu.async_copy(x_hbm.at[src_v.at[pl.ds(jj * K + KA, K - KA)]],
                         rows_v.at[t].at[pl.ds(KA, K - KA)], gsem2.at[t])

    def gather_wait(jj, t):
        pltpu.make_async_copy(x_hbm.at[src_v.at[pl.ds(jj * K, KA)]],
                              rows_v.at[t].at[pl.ds(0, KA)],
                              gsem.at[t]).wait()
        pltpu.make_async_copy(x_hbm.at[src_v.at[pl.ds(jj * K + KA, K - KA)]],
                              rows_v.at[t].at[pl.ds(KA, K - KA)],
                              gsem2.at[t]).wait()

    def scatter_start(jj, t):
        pltpu.make_async_copy(dst_hbm.at[wid, jj], dst_r.at[t],
                              dsem.at[t]).wait()
        pltpu.async_copy(rows_v.at[t], acc.at[dst_r.at[t]], ssem.at[t],
                         add=True)

    def scatter_wait(jj, t):
        pltpu.make_async_copy(rows_v.at[t], acc.at[dst_r.at[t]],
                              ssem.at[t]).wait()

    for t in range(NBUF):
        gather_start(t, t)

    @pl.loop(0, NCHUNK, step=NBUF)
    def _grp(j):
        for t in range(NBUF):
            jj = j + t
            gather_wait(jj, t)

            @pl.loop(0, K, step=8)
            def _scale(i):
                for u in range(8):
                    wv = plsc.load_gather(
                        w_v, [jnp.full((16,), jj * K + i + u, jnp.int32)])
                    for c in range(0, D, 16):
                        rows_v[t, i + u, pl.ds(c, 16)] = (
                            rows_v[t, i + u, pl.ds(c, 16)] * wv)

            scatter_start(jj, t)

            pt = (t - 1) % NBUF
            pj = jj - 1

            @pl.when(jnp.logical_and(pj >= 0, pj + NBUF < NCHUNK))
            def _refill():
                scatter_wait(pj, pt)
                gather_start(pj + NBUF, pt)

    for t in range(NBUF):
        scatter_wait(NCHUNK - NBUF + t, t)

    plsc.subcore_barrier()

    pltpu.sync_copy(acc.at[pl.ds(sid * RPS, RPS)],
                    out_hbm.at[cid, pl.ds(sid * RPS, RPS)])


_BLK = 2000


def _gru_body(p_ref, x_ref, wih_ref, whh_ref, bih_ref, bhh_ref, o_ref):
    aggr = p_ref[0] + p_ref[1]
    x = x_ref[...]
    gi = jnp.dot(aggr, wih_ref[...], precision=lax.Precision.HIGHEST,
                 preferred_element_type=jnp.float32) + bih_ref[...]
    gh = jnp.dot(x, whh_ref[...], precision=lax.Precision.HIGHEST,
                 preferred_element_type=jnp.float32) + bhh_ref[...]
    r = jax.nn.sigmoid(gi[:, :D] + gh[:, :D])
    z = jax.nn.sigmoid(gi[:, D:2 * D] + gh[:, D:2 * D])
    n = jnp.tanh(gi[:, 2 * D:] + r * gh[:, 2 * D:])
    o_ref[...] = (1.0 - z) * n + z * x


def _tc_gru(p, x, wihT, whhT, bih, bhh):
    return pl.pallas_call(
        _gru_body,
        out_shape=jax.ShapeDtypeStruct((N, D), jnp.float32),
        grid=(N // _BLK,),
        in_specs=[
            pl.BlockSpec((NC, _BLK, D), lambda i: (0, i, 0)),
            pl.BlockSpec((_BLK, D), lambda i: (i, 0)),
            pl.BlockSpec((D, 3 * D), lambda i: (0, 0)),
            pl.BlockSpec((D, 3 * D), lambda i: (0, 0)),
            pl.BlockSpec((1, 3 * D), lambda i: (0, 0)),
            pl.BlockSpec((1, 3 * D), lambda i: (0, 0)),
        ],
        out_specs=pl.BlockSpec((_BLK, D), lambda i: (i, 0)),
    )(p, x, wihT, whhT, bih, bhh)


def kernel(x, edge_index, edge_weight,
           W_ih1, W_hh1, b_ih1, b_hh1, W_ih2, W_hh2, b_ih2, b_hh2):
    src = edge_index[0].reshape(NW, EPW)
    dst = edge_index[1].reshape(NW, NCHUNK, K)
    w = edge_weight.reshape(NW, EPW)

    wih1T, whh1T = W_ih1.T, W_hh1.T
    wih2T, whh2T = W_ih2.T, W_hh2.T
    bih1, bhh1 = b_ih1.reshape(1, 3 * D), b_hh1.reshape(1, 3 * D)
    bih2, bhh2 = b_ih2.reshape(1, 3 * D), b_hh2.reshape(1, 3 * D)

    p1 = _sc_aggregate(x, src, dst, w)
    h = _tc_gru(p1, x, wih1T, whh1T, bih1, bhh1)
    p2 = _sc_aggregate(h, src, dst, w)
    return _tc_gru(p2, h, wih2T, whh2T, bih2, bhh2)


# speedup vs baseline: 3.5361x; 1.0021x over previous
import dataclasses
import functools

import jax
import jax.numpy as jnp
from jax import lax
from jax.experimental import pallas as pl
from jax.experimental.pallas import tpu as pltpu
from jax.experimental.pallas import tpu_sc as plsc

N = 10000
E = 320000
D = 128

NC = 2
NS = 16
NW = NC * NS
EPW = E // NW
K = 40
NCHUNK = EPW // K
NBUF = 5
NPAD = 10240
RPS = NPAD // NS

_mesh = plsc.VectorSubcoreMesh(core_axis_name="c", subcore_axis_name="s")

_sc_params = pltpu.CompilerParams()
if "needs_layout_passes" in pltpu.CompilerParams.__dataclass_fields__:
    _sc_params = dataclasses.replace(_sc_params, needs_layout_passes=False)


@functools.partial(
    pl.kernel,
    out_type=jax.ShapeDtypeStruct((NC, NPAD, D), jnp.float32),
    mesh=_mesh,
    compiler_params=_sc_params,
    scratch_types=[
        pltpu.VMEM_SHARED((NPAD, D), jnp.float32),
        pltpu.VMEM((EPW,), jnp.int32),
        pltpu.VMEM((NBUF, K), jnp.int32),
        pltpu.VMEM((EPW,), jnp.float32),
        pltpu.VMEM((NBUF, K, D), jnp.float32),
        pltpu.SemaphoreType.DMA((NBUF,)),
        pltpu.SemaphoreType.DMA((NBUF,)),
        pltpu.SemaphoreType.DMA((NBUF,)),
    ],
)
def _sc_aggregate(x_hbm, src_hbm, dst_hbm, w_hbm, out_hbm,
                  acc, src_v, dst_r, w_v, rows_v, gsem, ssem, dsem):
    cid = lax.axis_index("c")
    sid = lax.axis_index("s")
    wid = sid * NC + cid

    zbuf = rows_v.at[0]

    @pl.loop(0, K)
    def _zrow(i):
        for c in range(0, D, 16):
            zbuf[i, pl.ds(c, 16)] = jnp.zeros((16,), jnp.float32)

    pltpu.async_copy(src_hbm.at[wid], src_v, gsem.at[0])
    pltpu.async_copy(w_hbm.at[wid], w_v, dsem.at[0])

    @pl.loop(0, RPS, step=K)
    def _zcopy(r):
        pltpu.async_copy(zbuf, acc.at[pl.ds(sid * RPS + r, K)], ssem.at[0])

    @pl.loop(0, RPS, step=K)
    def _zwait(r):
        pltpu.make_async_copy(zbuf, acc.at[pl.ds(sid * RPS + r, K)],
                              ssem.at[0]).wait()

    pltpu.make_async_copy(src_hbm.at[wid], src_v, gsem.at[0]).wait()
    pltpu.make_async_copy(w_hbm.at[wid], w_v, dsem.at[0]).wait()

    plsc.subcore_barrier()

    def gather_start(jj, t):
        pltpu.async_copy(dst_hbm.at[wid, jj], dst_r.at[t], dsem.at[t])
        pltpu.async_copy(x_hbm.at[src_v.at[pl.ds(jj * K, K)]],
                         rows_v.at[t], gsem.at[t])

    def gather_wait(jj, t):
        pltpu.make_async_copy(x_hbm.at[src_v.at[pl.ds(jj * K, K)]],
                              rows_v.at[t], gsem.at[t]).wait()

    def scatter_start(jj, t):
        pltpu.make_async_copy(dst_hbm.at[wid, jj], dst_r.at[t],
                              dsem.at[t]).wait()
        pltpu.async_copy(rows_v.at[t], acc.at[dst_r.at[t]], ssem.at[t],
                         add=True)

    def scatter_wait(jj, t):
        pltpu.make_async_copy(rows_v.at[t], acc.at[dst_r.at[t]],
                              ssem.at[t]).wait()

    for t in range(NBUF):
        gather_start(t, t)

    @pl.loop(0, NCHUNK, step=NBUF)
    def _grp(j):
        for t in range(NBUF):
            jj = j + t
            gather_wait(jj, t)

            @pl.loop(0, K, step=8)
            def _scale(i):
                for u in range(8):
                    wv = plsc.load_gather(
                        w_v, [jnp.full((16,), jj * K + i + u, jnp.int32)])
                    for c in range(0, D, 16):
                        rows_v[t, i + u, pl.ds(c, 16)] = (
                            rows_v[t, i + u, pl.ds(c, 16)] * wv)

            scatter_start(jj, t)

            pt = (t - 1) % NBUF
            pj = jj - 1

            @pl.when(jnp.logical_and(pj >= 0, pj + NBUF < NCHUNK))
            def _refill():
                scatter_wait(pj, pt)
                gather_start(pj + NBUF, pt)

    for t in range(NBUF):
        scatter_wait(NCHUNK - NBUF + t, t)

    plsc.subcore_barrier()

    pltpu.sync_copy(acc.at[pl.ds(sid * RPS, RPS)],
                    out_hbm.at[cid, pl.ds(sid * RPS, RPS)])


_BLK = 2000


def _gru_body(p_ref, x_ref, wih_ref, whh_ref, bih_ref, bhh_ref, o_ref):
    aggr = p_ref[0] + p_ref[1]
    x = x_ref[...]
    gi = jnp.dot(aggr, wih_ref[...], precision=lax.Precision.HIGHEST,
                 preferred_element_type=jnp.float32) + bih_ref[...]
    gh = jnp.dot(x, whh_ref[...], precision=lax.Precision.HIGHEST,
                 preferred_element_type=jnp.float32) + bhh_ref[...]
    r = jax.nn.sigmoid(gi[:, :D] + gh[:, :D])
    z = jax.nn.sigmoid(gi[:, D:2 * D] + gh[:, D:2 * D])
    n = jnp.tanh(gi[:, 2 * D:] + r * gh[:, 2 * D:])
    o_ref[...] = (1.0 - z) * n + z * x


def _tc_gru(p, x, wihT, whhT, bih, bhh):
    return pl.pallas_call(
        _gru_body,
        out_shape=jax.ShapeDtypeStruct((N, D), jnp.float32),
        grid=(N // _BLK,),
        in_specs=[
            pl.BlockSpec((NC, _BLK, D), lambda i: (0, i, 0)),
            pl.BlockSpec((_BLK, D), lambda i: (i, 0)),
            pl.BlockSpec((D, 3 * D), lambda i: (0, 0)),
            pl.BlockSpec((D, 3 * D), lambda i: (0, 0)),
            pl.BlockSpec((1, 3 * D), lambda i: (0, 0)),
            pl.BlockSpec((1, 3 * D), lambda i: (0, 0)),
        ],
        out_specs=pl.BlockSpec((_BLK, D), lambda i: (i, 0)),
    )(p, x, wihT, whhT, bih, bhh)


def kernel(x, edge_index, edge_weight,
           W_ih1, W_hh1, b_ih1, b_hh1, W_ih2, W_hh2, b_ih2, b_hh2):
    src = edge_index[0].reshape(NW, EPW)
    dst = edge_index[1].reshape(NW, NCHUNK, K)
    w = edge_weight.reshape(NW, EPW)

    wih1T, whh1T = W_ih1.T, W_hh1.T
    wih2T, whh2T = W_ih2.T, W_hh2.T
    bih1, bhh1 = b_ih1.reshape(1, 3 * D), b_hh1.reshape(1, 3 * D)
    bih2, bhh2 = b_ih2.reshape(1, 3 * D), b_hh2.reshape(1, 3 * D)

    p1 = _sc_aggregate(x, src, dst, w)
    h = _tc_gru(p1, x, wih1T, whh1T, bih1, bhh1)
    p2 = _sc_aggregate(h, src, dst, w)
    return _tc_gru(p2, h, wih2T, whh2T, bih2, bhh2)
